# baseline (speedup 1.0000x reference)
import jax
import jax.numpy as jnp
from jax.experimental import pallas as pl

_BB = 16


def _fused_kernel(x_ref, v_ref, m_ref, xo_ref, vo_ref, mo_ref):
    xo_ref[...] = x_ref[...]
    vo_ref[...] = v_ref[...]
    mo_ref[...] = m_ref[...] != 0.0


def kernel(x, v, mask):
    B, C, P = x.shape
    Vc = v.shape[1]
    m2 = mask.reshape(B, P)
    grid = (B // _BB,)
    xo, vo, mo = pl.pallas_call(
        _fused_kernel,
        grid=grid,
        in_specs=[
            pl.BlockSpec((_BB, C, P), lambda i: (i, 0, 0)),
            pl.BlockSpec((_BB, Vc, P), lambda i: (i, 0, 0)),
            pl.BlockSpec((_BB, P), lambda i: (i, 0)),
        ],
        out_specs=[
            pl.BlockSpec((_BB, C, P), lambda i: (i, 0, 0)),
            pl.BlockSpec((_BB, Vc, P), lambda i: (i, 0, 0)),
            pl.BlockSpec((_BB, P), lambda i: (i, 0)),
        ],
        out_shape=[
            jax.ShapeDtypeStruct((B, C, P), x.dtype),
            jax.ShapeDtypeStruct((B, Vc, P), v.dtype),
            jax.ShapeDtypeStruct((B, P), jnp.bool_),
        ],
    )(x, v, m2)
    return (xo, vo, mo.reshape(B, 1, P))


# speedup vs baseline: 1.0308x; 1.0308x over previous
import jax
import jax.numpy as jnp
from jax.experimental import pallas as pl

_BB = 32


def _fused_kernel(x_ref, v_ref, m_ref, xo_ref, vo_ref, mo_ref):
    xo_ref[...] = x_ref[...]
    vo_ref[...] = v_ref[...]
    mo_ref[...] = m_ref[...] != 0.0


def kernel(x, v, mask):
    B, C, P = x.shape
    Vc = v.shape[1]
    m2 = mask.reshape(B, P)
    grid = (B // _BB,)
    xo, vo, mo = pl.pallas_call(
        _fused_kernel,
        grid=grid,
        in_specs=[
            pl.BlockSpec((_BB, C, P), lambda i: (i, 0, 0)),
            pl.BlockSpec((B, Vc, P), lambda i: (0, 0, 0)),
            pl.BlockSpec((B, P), lambda i: (0, 0)),
        ],
        out_specs=[
            pl.BlockSpec((_BB, C, P), lambda i: (i, 0, 0)),
            pl.BlockSpec((B, Vc, P), lambda i: (0, 0, 0)),
            pl.BlockSpec((B, P), lambda i: (0, 0)),
        ],
        out_shape=[
            jax.ShapeDtypeStruct((B, C, P), x.dtype),
            jax.ShapeDtypeStruct((B, Vc, P), v.dtype),
            jax.ShapeDtypeStruct((B, P), jnp.bool_),
        ],
    )(x, v, m2)
    return (xo, vo, mo.reshape(B, 1, P))


# speedup vs baseline: 1.0358x; 1.0049x over previous
import jax
import jax.numpy as jnp
from jax.experimental import pallas as pl

_BB = 32


def _fused_kernel(x_ref, v_ref, m_ref, xo_ref, vo_ref, mo_ref):
    xo_ref[...] = x_ref[...]
    vo_ref[...] = v_ref[...]
    mo_ref[...] = m_ref[...] != 0.0


def kernel(x, v, mask):
    B, C, P = x.shape
    Vc = v.shape[1]
    m2 = mask.reshape(B, P)
    grid = (B // _BB,)
    xo, vo, mo = pl.pallas_call(
        _fused_kernel,
        grid=grid,
        in_specs=[
            pl.BlockSpec((_BB, C, P), lambda i: (i, 0, 0)),
            pl.BlockSpec((_BB, Vc, P), lambda i: (i, 0, 0)),
            pl.BlockSpec((_BB, P), lambda i: (i, 0)),
        ],
        out_specs=[
            pl.BlockSpec((_BB, C, P), lambda i: (i, 0, 0)),
            pl.BlockSpec((_BB, Vc, P), lambda i: (i, 0, 0)),
            pl.BlockSpec((_BB, P), lambda i: (i, 0)),
        ],
        out_shape=[
            jax.ShapeDtypeStruct((B, C, P), x.dtype),
            jax.ShapeDtypeStruct((B, Vc, P), v.dtype),
            jax.ShapeDtypeStruct((B, P), jnp.bool_),
        ],
    )(x, v, m2)
    return (xo, vo, mo.reshape(B, 1, P))
